# baseline (speedup 1.0000x reference)
import jax
import jax.numpy as jnp
from jax.experimental import pallas as pl
from jax.experimental.pallas import tpu as pltpu

_NUM_CODES = 1024
_CODE_DIM = 64
_COMMITMENT_COST = 0.25
_BLOCK_ROWS = 4


def _vq_block_kernel(z_ref, emb_ref, zq_ref, idx_ref, loss_ref):
    blk = _BLOCK_ROWS * 1024
    z = z_ref[...].reshape(blk, _CODE_DIM)
    emb = emb_ref[...]
    z_sq = jnp.sum(z * z, axis=1)
    e_sq = jnp.sum(emb * emb, axis=1)
    emb_aug = jnp.concatenate([-2.0 * emb, e_sq[:, None]], axis=1)
    z_aug = jnp.concatenate(
        [z, jnp.ones((blk, 1), jnp.float32)], axis=1)
    u = jax.lax.dot_general(
        z_aug, emb_aug, (((1,), (1,)), ((), ())),
        preferred_element_type=jnp.float32)
    umin = jnp.min(u, axis=1)
    iota = jax.lax.broadcasted_iota(jnp.int32, (1, _NUM_CODES), 1).astype(
        jnp.float32)
    idx = jnp.min(
        jnp.where(u == umin[:, None], iota, float(_NUM_CODES)), axis=1
    ).astype(jnp.int32)

    zq_ref[...] = z_ref[...]
    idx_ref[0, :, :] = idx.reshape(_BLOCK_ROWS, 1024)
    loss_ref[0, 0, 0] = jnp.sum(umin + z_sq)


def kernel(z, embedding):
    rows = z.shape[0]
    grid = rows // _BLOCK_ROWS

    zq, idx3, loss_sum = pl.pallas_call(
        _vq_block_kernel,
        grid=(grid,),
        in_specs=[
            pl.BlockSpec((_BLOCK_ROWS, 1024, _CODE_DIM), lambda i: (i, 0, 0)),
            pl.BlockSpec((_NUM_CODES, _CODE_DIM), lambda i: (0, 0)),
        ],
        out_specs=[
            pl.BlockSpec((_BLOCK_ROWS, 1024, _CODE_DIM), lambda i: (i, 0, 0)),
            pl.BlockSpec((1, _BLOCK_ROWS, 1024), lambda i: (i, 0, 0)),
            pl.BlockSpec((1, 1, 1), lambda i: (i, 0, 0),
                         memory_space=pltpu.SMEM),
        ],
        out_shape=[
            jax.ShapeDtypeStruct(z.shape, jnp.float32),
            jax.ShapeDtypeStruct((grid, _BLOCK_ROWS, 1024), jnp.int32),
            jax.ShapeDtypeStruct((grid, 1, 1), jnp.float32),
        ],
        compiler_params=pltpu.CompilerParams(
            dimension_semantics=("parallel",)),
    )(z, embedding)

    indices = idx3.reshape(z.shape[:-1])
    vq_loss = _COMMITMENT_COST * jnp.sum(loss_sum) / (rows * 1024 * _CODE_DIM)
    return (zq, indices, vq_loss)


# speedup vs baseline: 1.0139x; 1.0139x over previous
import jax
import jax.numpy as jnp
from jax.experimental import pallas as pl
from jax.experimental.pallas import tpu as pltpu

_NUM_CODES = 1024
_CODE_DIM = 64
_COMMITMENT_COST = 0.25
_BLOCK_ROWS = 4


def _vq_block_kernel(z_ref, emb_ref, zq_ref, idx_ref, loss_ref):
    blk = _BLOCK_ROWS * 1024
    z = z_ref[...].reshape(blk, _CODE_DIM)
    emb = emb_ref[...]
    z_sq = jnp.sum(z * z, axis=1, keepdims=True)
    e_sq = jnp.sum(emb * emb, axis=1)
    neg2_emb = -2.0 * emb
    scores2 = jax.lax.dot_general(
        z, neg2_emb, (((1,), (1,)), ((), ())),
        preferred_element_type=jnp.float32)
    u = (z_sq + scores2) + e_sq[None, :]
    umin = jnp.min(u, axis=1)
    iota = jax.lax.broadcasted_iota(jnp.int32, (1, _NUM_CODES), 1).astype(
        jnp.float32)
    idx = jnp.min(
        jnp.where(u == umin[:, None], iota, float(_NUM_CODES)), axis=1
    ).astype(jnp.int32)

    zq_ref[...] = z_ref[...]
    idx_ref[0, :, :] = idx.reshape(_BLOCK_ROWS, 1024)
    loss_ref[0, 0, 0] = jnp.sum(umin)


def kernel(z, embedding):
    rows = z.shape[0]
    grid = rows // _BLOCK_ROWS

    zq, idx3, loss_sum = pl.pallas_call(
        _vq_block_kernel,
        grid=(grid,),
        in_specs=[
            pl.BlockSpec((_BLOCK_ROWS, 1024, _CODE_DIM), lambda i: (i, 0, 0)),
            pl.BlockSpec((_NUM_CODES, _CODE_DIM), lambda i: (0, 0)),
        ],
        out_specs=[
            pl.BlockSpec((_BLOCK_ROWS, 1024, _CODE_DIM), lambda i: (i, 0, 0)),
            pl.BlockSpec((1, _BLOCK_ROWS, 1024), lambda i: (i, 0, 0)),
            pl.BlockSpec((1, 1, 1), lambda i: (i, 0, 0),
                         memory_space=pltpu.SMEM),
        ],
        out_shape=[
            jax.ShapeDtypeStruct(z.shape, jnp.float32),
            jax.ShapeDtypeStruct((grid, _BLOCK_ROWS, 1024), jnp.int32),
            jax.ShapeDtypeStruct((grid, 1, 1), jnp.float32),
        ],
        compiler_params=pltpu.CompilerParams(
            dimension_semantics=("parallel",)),
    )(z, embedding)

    indices = idx3.reshape(z.shape[:-1])
    vq_loss = _COMMITMENT_COST * jnp.sum(loss_sum) / (rows * 1024 * _CODE_DIM)
    return (zq, indices, vq_loss)


# speedup vs baseline: 1.0832x; 1.0684x over previous
import jax
import jax.numpy as jnp
from jax.experimental import pallas as pl
from jax.experimental.pallas import tpu as pltpu

_NUM_CODES = 1024
_CODE_DIM = 64
_COMMITMENT_COST = 0.25
_BLOCK_ROWS = 4


def _vq_block_kernel(z_ref, emb_ref, idx_ref, loss_ref):
    blk = _BLOCK_ROWS * 1024
    z = z_ref[...].reshape(blk, _CODE_DIM)
    emb = emb_ref[...]
    z_sq = jnp.sum(z * z, axis=1, keepdims=True)
    e_sq = jnp.sum(emb * emb, axis=1)
    neg2_emb = -2.0 * emb
    scores2 = jax.lax.dot_general(
        z, neg2_emb, (((1,), (1,)), ((), ())),
        preferred_element_type=jnp.float32)
    u = (z_sq + scores2) + e_sq[None, :]
    umin = jnp.min(u, axis=1)
    iota = jax.lax.broadcasted_iota(jnp.int32, (1, _NUM_CODES), 1).astype(
        jnp.float32)
    idx = jnp.min(
        jnp.where(u == umin[:, None], iota, float(_NUM_CODES)), axis=1
    ).astype(jnp.int32)

    idx_ref[0, :, :] = idx.reshape(_BLOCK_ROWS, 1024)
    loss_ref[0, 0, 0] = jnp.sum(umin)


def kernel(z, embedding):
    rows = z.shape[0]
    grid = rows // _BLOCK_ROWS

    idx3, loss_sum = pl.pallas_call(
        _vq_block_kernel,
        grid=(grid,),
        in_specs=[
            pl.BlockSpec((_BLOCK_ROWS, 1024, _CODE_DIM), lambda i: (i, 0, 0)),
            pl.BlockSpec((_NUM_CODES, _CODE_DIM), lambda i: (0, 0)),
        ],
        out_specs=[
            pl.BlockSpec((1, _BLOCK_ROWS, 1024), lambda i: (i, 0, 0)),
            pl.BlockSpec((1, 1, 1), lambda i: (i, 0, 0),
                         memory_space=pltpu.SMEM),
        ],
        out_shape=[
            jax.ShapeDtypeStruct((grid, _BLOCK_ROWS, 1024), jnp.int32),
            jax.ShapeDtypeStruct((grid, 1, 1), jnp.float32),
        ],
        compiler_params=pltpu.CompilerParams(
            dimension_semantics=("parallel",)),
    )(z, embedding)

    indices = idx3.reshape(z.shape[:-1])
    vq_loss = _COMMITMENT_COST * jnp.sum(loss_sum) / (rows * 1024 * _CODE_DIM)
    return (z, indices, vq_loss)


# speedup vs baseline: 2.1775x; 2.0103x over previous
import jax
import jax.numpy as jnp
from jax.experimental import pallas as pl
from jax.experimental.pallas import tpu as pltpu

_NUM_CODES = 1024
_CODE_DIM = 64
_COMMITMENT_COST = 0.25
_BLOCK_ROWS = 4


def _vq_block_kernel(zt_ref, emb_ref, idx_ref, loss_ref):
    emb = emb_ref[...]
    neg2_emb = -2.0 * emb
    e_sq_col = jnp.sum(emb * emb, axis=1)[:, None]
    iota_col = jax.lax.broadcasted_iota(
        jnp.int32, (_NUM_CODES, 1), 0).astype(jnp.float32)

    total = jnp.float32(0.0)
    for r in range(_BLOCK_ROWS):
        zt = zt_ref[r]
        z_sq_row = jnp.sum(zt * zt, axis=0)[None, :]
        s2 = jax.lax.dot_general(
            neg2_emb, zt, (((1,), (0,)), ((), ())),
            preferred_element_type=jnp.float32)
        d = (z_sq_row + s2) + e_sq_col
        dmin = jnp.min(d, axis=0)
        w = jnp.where(d == dmin[None, :], iota_col, float(_NUM_CODES))
        idx_ref[0, r, :] = jnp.min(w, axis=0).astype(jnp.int32)
        total = total + jnp.sum(dmin)

    loss_ref[0, 0, 0] = total


def kernel(z, embedding):
    rows = z.shape[0]
    grid = rows // _BLOCK_ROWS
    zt = jnp.transpose(z, (0, 2, 1))

    idx3, loss_sum = pl.pallas_call(
        _vq_block_kernel,
        grid=(grid,),
        in_specs=[
            pl.BlockSpec((_BLOCK_ROWS, _CODE_DIM, 1024), lambda i: (i, 0, 0)),
            pl.BlockSpec((_NUM_CODES, _CODE_DIM), lambda i: (0, 0)),
        ],
        out_specs=[
            pl.BlockSpec((1, _BLOCK_ROWS, 1024), lambda i: (i, 0, 0)),
            pl.BlockSpec((1, 1, 1), lambda i: (i, 0, 0),
                         memory_space=pltpu.SMEM),
        ],
        out_shape=[
            jax.ShapeDtypeStruct((grid, _BLOCK_ROWS, 1024), jnp.int32),
            jax.ShapeDtypeStruct((grid, 1, 1), jnp.float32),
        ],
        compiler_params=pltpu.CompilerParams(
            dimension_semantics=("parallel",)),
    )(zt, embedding)

    indices = idx3.reshape(z.shape[:-1])
    vq_loss = _COMMITMENT_COST * jnp.sum(loss_sum) / (rows * 1024 * _CODE_DIM)
    return (z, indices, vq_loss)


# speedup vs baseline: 2.4817x; 1.1397x over previous
import jax
import jax.numpy as jnp
from jax.experimental import pallas as pl
from jax.experimental.pallas import tpu as pltpu

_NUM_CODES = 1024
_CODE_DIM = 64
_COMMITMENT_COST = 0.25
_BLOCK_ROWS = 8


def _vq_block_kernel(zt_ref, emb_ref, zq_ref, idx_ref, loss_ref):
    emb = emb_ref[...]
    neg2_emb = -2.0 * emb
    e_sq_col = jnp.sum(emb * emb, axis=1)[:, None]
    iota_col = jax.lax.broadcasted_iota(
        jnp.int32, (_NUM_CODES, 1), 0).astype(jnp.float32)

    total = jnp.float32(0.0)
    for r in range(_BLOCK_ROWS):
        zt = zt_ref[r]
        z_sq_row = jnp.sum(zt * zt, axis=0)[None, :]
        s2 = jax.lax.dot_general(
            neg2_emb, zt, (((1,), (0,)), ((), ())),
            preferred_element_type=jnp.float32)
        d = (z_sq_row + s2) + e_sq_col
        dmin = jnp.min(d, axis=0)
        w = jnp.where(d == dmin[None, :], iota_col, float(_NUM_CODES))
        idx_ref[r, :] = jnp.min(w, axis=0).astype(jnp.int32)
        total = total + jnp.sum(dmin)

    zq_ref[...] = zt_ref[...]
    loss_ref[0, 0, 0] = total


def kernel(z, embedding):
    rows = z.shape[0]
    grid = rows // _BLOCK_ROWS
    zt = jnp.transpose(z, (0, 2, 1))

    zq_t, indices, loss_sum = pl.pallas_call(
        _vq_block_kernel,
        grid=(grid,),
        in_specs=[
            pl.BlockSpec((_BLOCK_ROWS, _CODE_DIM, 1024), lambda i: (i, 0, 0)),
            pl.BlockSpec((_NUM_CODES, _CODE_DIM), lambda i: (0, 0)),
        ],
        out_specs=[
            pl.BlockSpec((_BLOCK_ROWS, _CODE_DIM, 1024), lambda i: (i, 0, 0)),
            pl.BlockSpec((_BLOCK_ROWS, 1024), lambda i: (i, 0)),
            pl.BlockSpec((1, 1, 1), lambda i: (i, 0, 0),
                         memory_space=pltpu.SMEM),
        ],
        out_shape=[
            jax.ShapeDtypeStruct(zt.shape, jnp.float32),
            jax.ShapeDtypeStruct((rows, 1024), jnp.int32),
            jax.ShapeDtypeStruct((grid, 1, 1), jnp.float32),
        ],
        compiler_params=pltpu.CompilerParams(
            dimension_semantics=("parallel",)),
    )(zt, embedding)

    z_q_st = jnp.transpose(zq_t, (0, 2, 1))
    vq_loss = _COMMITMENT_COST * jnp.sum(loss_sum) / (rows * 1024 * _CODE_DIM)
    return (z_q_st, indices, vq_loss)
